# baseline (speedup 1.0000x reference)
import functools

import jax
import jax.numpy as jnp
from jax import lax
from jax.experimental import pallas as pl
from jax.experimental.pallas import tpu as pltpu
from jax.experimental.pallas import tpu_sc as plsc

DIM = 16
LANES = 16
NC = 2
NS = 16
NW = NC * NS
CHUNK = 1024


@functools.lru_cache(maxsize=None)
def _softalign_sc(n_rows):
    bpw = n_rows // NW
    nch = bpw // CHUNK
    mesh = plsc.VectorSubcoreMesh(core_axis_name="c", subcore_axis_name="s")

    @functools.partial(
        pl.kernel,
        mesh=mesh,
        compiler_params=pltpu.CompilerParams(
            needs_layout_passes=False, use_tc_tiling_on_sc=False
        ),
        out_type=jax.ShapeDtypeStruct((n_rows, DIM), jnp.float32),
        scratch_types=[
            pltpu.VMEM((bpw,), jnp.int32),
            pltpu.VMEM((CHUNK, DIM), jnp.float32),
            pltpu.SemaphoreType.DMA,
        ],
    )
    def k(idx_hbm, proj_hbm, out_hbm, idx_v, buf_v, sem):
        c = lax.axis_index("c")
        s = lax.axis_index("s")
        wid = s * NC + c
        row0 = wid * bpw
        pltpu.sync_copy(idx_hbm.at[pl.ds(row0, bpw)], idx_v)

        for j in range(nch):
            pltpu.async_copy(
                proj_hbm.at[idx_v.at[pl.ds(j * CHUNK, CHUNK)]], buf_v, sem
            ).wait()

            def body(i, carry):
                e = jnp.exp(buf_v[i, :])
                s = lax.broadcast(jnp.sum(e), (LANES,))
                buf_v[i, :] = e / s
                return carry

            lax.fori_loop(0, CHUNK, body, 0)
            pltpu.sync_copy(buf_v, out_hbm.at[pl.ds(row0 + j * CHUNK, CHUNK)])

    return k


def kernel(input, proj):
    n_rows = input.shape[0] * input.shape[1]
    idx = input.reshape(n_rows).astype(jnp.int32)
    out = _softalign_sc(n_rows)(idx, proj)
    return out.reshape(input.shape[0], input.shape[1], DIM)


# speedup vs baseline: 1.3254x; 1.3254x over previous
import functools

import jax
import jax.numpy as jnp
from jax import lax
from jax.experimental import pallas as pl
from jax.experimental.pallas import tpu as pltpu
from jax.experimental.pallas import tpu_sc as plsc

DIM = 16
LANES = 16
NC = 2
NS = 16
NW = NC * NS
CHUNK = 1024
NBUF = 4
PRE = 3
UNROLL = 8


@functools.lru_cache(maxsize=None)
def _softalign_sc(batch, fields):
    n_rows = batch * fields
    bpw = n_rows // NW
    nch = bpw // CHUNK
    mesh = plsc.VectorSubcoreMesh(core_axis_name="c", subcore_axis_name="s")

    @functools.partial(
        pl.kernel,
        mesh=mesh,
        compiler_params=pltpu.CompilerParams(
            needs_layout_passes=False, use_tc_tiling_on_sc=False
        ),
        out_type=jax.ShapeDtypeStruct((n_rows, DIM), jnp.float32),
        scratch_types=(
            [
                pltpu.VMEM((bpw,), jnp.int32),
                pltpu.VMEM((NBUF, CHUNK, DIM), jnp.float32),
            ]
            + [pltpu.SemaphoreType.DMA] * (2 * NBUF)
        ),
    )
    def k(idx_hbm, proj_hbm, out_hbm, idx_v, bufs_v, *sems):
        gsem = sems[:NBUF]
        osem = sems[NBUF:]
        idx_flat = idx_hbm
        out_flat = out_hbm

        c = lax.axis_index("c")
        s = lax.axis_index("s")
        wid = s * NC + c
        row0 = wid * bpw
        pltpu.sync_copy(idx_flat.at[pl.ds(row0, bpw)], idx_v)

        def fire_gather(t):
            b = t % NBUF
            return pltpu.async_copy(
                proj_hbm.at[idx_v.at[pl.ds(t * CHUNK, CHUNK)]],
                bufs_v.at[b],
                gsem[b],
            )

        pending_g = {}
        pending_o = {}
        for t in range(min(PRE, nch)):
            pending_g[t] = fire_gather(t)

        for j in range(nch):
            b = j % NBUF
            pending_g.pop(j).wait()

            def body(g, carry):
                i0 = g * UNROLL
                es = [jnp.exp(bufs_v[b, i0 + r, :]) for r in range(UNROLL)]
                ss = [lax.broadcast(jnp.sum(e), (LANES,)) for e in es]
                for r in range(UNROLL):
                    bufs_v[b, i0 + r, :] = es[r] / ss[r]
                return carry

            lax.fori_loop(0, CHUNK // UNROLL, body, 0)

            pending_o[j] = pltpu.async_copy(
                bufs_v.at[b],
                out_flat.at[pl.ds(row0 + j * CHUNK, CHUNK)],
                osem[b],
            )
            t = j + PRE
            if t < nch:
                if t - NBUF >= 0:
                    pending_o.pop(t - NBUF).wait()
                pending_g[t] = fire_gather(t)

        for j in sorted(pending_o):
            pending_o[j].wait()

    return k


def kernel(input, proj):
    batch, fields = input.shape
    n_rows = batch * fields
    out = _softalign_sc(batch, fields)(
        input.reshape(n_rows).astype(jnp.int32), proj
    )
    return out.reshape(batch, fields, DIM)


# speedup vs baseline: 2.3484x; 1.7719x over previous
import functools

import jax
import jax.numpy as jnp
from jax import lax
from jax.experimental import pallas as pl
from jax.experimental.pallas import tpu as pltpu
from jax.experimental.pallas import tpu_sc as plsc

DIM = 16
LANES = 16
NC = 2
NS = 16
NW = NC * NS
CHUNK = 1024
NG = 2
NO = 2


@functools.lru_cache(maxsize=None)
def _softalign_sc(batch, fields):
    n_rows = batch * fields
    bpw = n_rows // NW
    nch = bpw // CHUNK
    cpf = batch // CHUNK
    mesh = plsc.VectorSubcoreMesh(core_axis_name="c", subcore_axis_name="s")

    @functools.partial(
        pl.kernel,
        mesh=mesh,
        compiler_params=pltpu.CompilerParams(
            needs_layout_passes=False, use_tc_tiling_on_sc=False
        ),
        out_type=jax.ShapeDtypeStruct((fields, DIM, batch), jnp.float32),
        scratch_types=(
            [
                pltpu.VMEM((nch, CHUNK), jnp.int32),
                pltpu.VMEM((NG, CHUNK, DIM), jnp.float32),
                pltpu.VMEM((NO, DIM, CHUNK), jnp.float32),
            ]
            + [pltpu.SemaphoreType.DMA] * (NG + NO)
        ),
    )
    def k(idxT_hbm, proj_hbm, outT_hbm, idx_v, gbuf_v, obuf_v, *sems):
        gsem = sems[:NG]
        osem = sems[NG:]

        c = lax.axis_index("c")
        s = lax.axis_index("s")
        wid = s * NC + c
        jc0 = wid * nch

        for t in range(nch):
            jc = jc0 + t
            f = jc // cpf
            b0 = (jc % cpf) * CHUNK
            pltpu.sync_copy(idxT_hbm.at[f, pl.ds(b0, CHUNK)], idx_v.at[t])

        def fire_gather(t):
            b = t % NG
            return pltpu.async_copy(
                proj_hbm.at[idx_v.at[t]], gbuf_v.at[b], gsem[b]
            )

        pending_g = {0: fire_gather(0)}
        pending_o = {}
        iota = lax.iota(jnp.int32, LANES)

        for j in range(nch):
            b = j % NG
            o = j % NO
            pending_g.pop(j).wait()
            if j + 1 < nch:
                pending_g[j + 1] = fire_gather(j + 1)
            if j - NO >= 0:
                pending_o.pop(j - NO).wait()

            def body(g, carry):
                rows = lax.broadcast(g * LANES, (LANES,)) + iota
                es = []
                for col in range(DIM):
                    cols = jnp.full((LANES,), col, jnp.int32)
                    v = plsc.load_gather(gbuf_v.at[b], [rows, cols])
                    es.append(jnp.exp(v))
                acc = es
                while len(acc) > 1:
                    acc = [acc[i] + acc[i + 1] for i in range(0, len(acc), 2)]
                r = 1.0 / acc[0]
                for col in range(DIM):
                    obuf_v[o, col, pl.ds(g * LANES, LANES)] = es[col] * r
                return carry

            lax.fori_loop(0, CHUNK // LANES, body, 0)

            jc = jc0 + j
            f = jc // cpf
            b0 = (jc % cpf) * CHUNK
            pending_o[j] = pltpu.async_copy(
                obuf_v.at[o],
                outT_hbm.at[f, :, pl.ds(b0, CHUNK)],
                osem[o],
            )

        for j in sorted(pending_o):
            pending_o[j].wait()

    return k


def kernel(input, proj):
    batch, fields = input.shape
    outT = _softalign_sc(batch, fields)(input.T.astype(jnp.int32), proj)
    return outT.transpose(2, 0, 1)
